# baseline (speedup 1.0000x reference)
import functools

import jax
import jax.numpy as jnp
from jax import lax
from jax.experimental import pallas as pl
from jax.experimental.pallas import tpu as pltpu
from jax.experimental.pallas import tpu_sc as plsc

NUM_ATOMIC = 119
DIM = 128
NC, NS = 2, 16
NW = NC * NS

CHUNK = 128
NBUF = 4
SB = 80
VTOK = 48
LANES = 16


@functools.partial(jax.jit, static_argnames=("b_total",))
def _sc_embed(idx2d, table, table_t_flat, b_total):
    b_per_w = b_total // NW
    n_chunks = b_per_w // CHUNK
    n_sb = n_chunks // SB
    rows_per_w = b_per_w // 128

    mesh = plsc.VectorSubcoreMesh(core_axis_name="c", subcore_axis_name="s")

    @functools.partial(
        pl.kernel,
        mesh=mesh,
        out_type=jax.ShapeDtypeStruct((b_total, DIM), jnp.float32),
        scratch_types=[
            pltpu.VMEM((2, SB, 128), jnp.int32),
            pltpu.VMEM((NBUF, CHUNK, DIM), jnp.float32),
            pltpu.VMEM((NUM_ATOMIC * DIM,), jnp.float32),
            pltpu.VMEM_SHARED((NUM_ATOMIC, DIM), jnp.float32),
            [pltpu.SemaphoreType.DMA] * NBUF,
            pltpu.SemaphoreType.DMA,
        ],
        compiler_params=pltpu.CompilerParams(needs_layout_passes=False),
    )
    def k(idx_hbm, table_hbm, tt_hbm, out_hbm, idx_sb, rows_v, table_tile,
          table_sp, ssem, isem):
        wid = lax.axis_index("s") * NC + lax.axis_index("c")
        row_base = wid * rows_per_w
        tok_base = wid * b_per_w

        @pl.when(lax.axis_index("s") == 0)
        def _():
            pltpu.sync_copy(table_hbm, table_sp)

        pltpu.sync_copy(tt_hbm, table_tile)
        plsc.subcore_barrier()

        lane_iota = lax.iota(jnp.int32, LANES)

        def idx_copy(sb, sbuf):
            return pltpu.make_async_copy(
                idx_hbm.at[pl.ds(row_base + sb * SB, SB)], idx_sb.at[sbuf], isem
            )

        def gather_copy(c, b, sbuf):
            return pltpu.make_async_copy(
                table_sp.at[idx_sb.at[sbuf, c, pl.ds(VTOK, CHUNK - VTOK)]],
                rows_v.at[b, pl.ds(VTOK, CHUNK - VTOK)],
                ssem[b],
            )

        def scatter_copy(g, b):
            return pltpu.make_async_copy(
                rows_v.at[b], out_hbm.at[pl.ds(tok_base + g * CHUNK, CHUNK)], ssem[b]
            )

        def build_rows(c, b, sbuf):
            bvec = jnp.full((LANES,), b, jnp.int32)
            for j16 in range(VTOK // LANES):
                tv = idx_sb[sbuf, c, pl.ds(j16 * LANES, LANES)]
                tokvec = jnp.int32(j16 * LANES) + lane_iota

                def per_d4(i, carry):
                    for r in range(4):
                        d = i * 4 + r
                        vals = plsc.load_gather(
                            table_tile, [d * NUM_ATOMIC + tv]
                        )
                        plsc.store_scatter(
                            rows_v,
                            [bvec, tokvec, jnp.full((LANES,), d, jnp.int32)],
                            vals,
                        )
                    return carry

                lax.fori_loop(0, DIM // 4, per_d4, 0)

        idx_copy(0, 0).start()
        idx_copy(0, 0).wait()

        def run_sb(sb, sbuf):
            gbase = sb * SB

            @pl.when(sb + 1 < n_sb)
            def _():
                idx_copy(sb + 1, 1 - sbuf).start()

            gather_copy(0, 0, sbuf).start()
            gather_copy(1, 1, sbuf).start()
            build_rows(0, 0, sbuf)
            build_rows(1, 1, sbuf)

            def body(q, carry):
                for b in range(NBUF):
                    c = q * NBUF + b
                    gather_copy(c, b, sbuf).wait()
                    scatter_copy(gbase + c, b).start()
                    nxt = c + 2
                    bn = (b + 2) % NBUF

                    @pl.when(nxt < SB)
                    def _():
                        @pl.when(nxt >= NBUF)
                        def _():
                            scatter_copy(gbase + nxt - NBUF, bn).wait()

                        gather_copy(nxt, bn, sbuf).start()
                        build_rows(nxt, bn, sbuf)

                return carry

            lax.fori_loop(0, SB // NBUF, body, 0)
            for b in range(NBUF):
                scatter_copy(gbase + SB - NBUF + b, b).wait()

            @pl.when(sb + 1 < n_sb)
            def _():
                idx_copy(sb + 1, 1 - sbuf).wait()

        def outer(sp, carry):
            run_sb(sp * 2, 0)
            run_sb(sp * 2 + 1, 1)
            return carry

        lax.fori_loop(0, n_sb // 2, outer, 0)

    return k(idx2d, table, table_t_flat)


def kernel(tokens, table):
    b, t = tokens.shape
    b_total = b * t
    idx2d = tokens.reshape(b_total // 128, 128).astype(jnp.int32)
    table_t_flat = table.T.reshape(-1)
    out = _sc_embed(idx2d, table, table_t_flat, b_total)
    return out.reshape(b, t, DIM)


# speedup vs baseline: 2.6977x; 2.6977x over previous
import functools

import jax
import jax.numpy as jnp
from jax import lax
from jax.experimental import pallas as pl
from jax.experimental.pallas import tpu as pltpu
from jax.experimental.pallas import tpu_sc as plsc

NUM_ATOMIC = 119
DIM = 128
NC, NS = 2, 16
NW = NC * NS

CHUNK = 128
NBUF = 4
SB = 80
VTOK = 48
LANES = 16


@functools.partial(jax.jit, static_argnames=("b_total",))
def _sc_embed(idx2d, table, table_t_flat, b_total):
    b_per_w = b_total // NW
    n_chunks = b_per_w // CHUNK
    n_sb = n_chunks // SB
    rows_per_w = b_per_w // 128

    mesh = plsc.VectorSubcoreMesh(core_axis_name="c", subcore_axis_name="s")

    @functools.partial(
        pl.kernel,
        mesh=mesh,
        out_type=jax.ShapeDtypeStruct((b_total, DIM), jnp.float32),
        scratch_types=[
            pltpu.VMEM((2, SB, 128), jnp.int32),
            pltpu.VMEM((NBUF, CHUNK, DIM), jnp.float32),
            pltpu.VMEM((NUM_ATOMIC * DIM,), jnp.float32),
            pltpu.VMEM_SHARED((NUM_ATOMIC, DIM), jnp.float32),
            [pltpu.SemaphoreType.DMA] * NBUF,
            pltpu.SemaphoreType.DMA,
        ],
        compiler_params=pltpu.CompilerParams(needs_layout_passes=False),
    )
    def k(idx_hbm, table_hbm, tt_hbm, out_hbm, idx_sb, rows_v, table_tile,
          table_sp, ssem, isem):
        wid = lax.axis_index("s") * NC + lax.axis_index("c")
        row_base = wid * rows_per_w
        tok_base = wid * b_per_w

        @pl.when(lax.axis_index("s") == 0)
        def _():
            pltpu.sync_copy(table_hbm, table_sp)

        pltpu.sync_copy(tt_hbm, table_tile)
        plsc.subcore_barrier()

        lane_iota = lax.iota(jnp.int32, LANES)
        offs = [
            (lane_iota + u * LANES) * NUM_ATOMIC for u in range(DIM // LANES)
        ]

        def idx_copy(sb, sbuf):
            return pltpu.make_async_copy(
                idx_hbm.at[pl.ds(row_base + sb * SB, SB)], idx_sb.at[sbuf], isem
            )

        def gather_copy(c, b, sbuf):
            return pltpu.make_async_copy(
                table_sp.at[idx_sb.at[sbuf, c, pl.ds(VTOK, CHUNK - VTOK)]],
                rows_v.at[b, pl.ds(VTOK, CHUNK - VTOK)],
                ssem[b],
            )

        def scatter_copy(g, b):
            return pltpu.make_async_copy(
                rows_v.at[b], out_hbm.at[pl.ds(tok_base + g * CHUNK, CHUNK)], ssem[b]
            )

        def build_rows(c, b, sbuf):
            def per_group(j16, carry):
                tvv = idx_sb[sbuf, c, pl.ds(j16 * LANES, LANES)]
                for l in range(LANES):
                    t = tvv[l]
                    j = j16 * LANES + l
                    for u in range(DIM // LANES):
                        vals = plsc.load_gather(table_tile, [offs[u] + t])
                        rows_v[b, j, pl.ds(u * LANES, LANES)] = vals
                return carry

            lax.fori_loop(0, VTOK // LANES, per_group, 0)

        idx_copy(0, 0).start()
        idx_copy(0, 0).wait()

        def run_sb(sb, sbuf):
            gbase = sb * SB

            @pl.when(sb + 1 < n_sb)
            def _():
                idx_copy(sb + 1, 1 - sbuf).start()

            gather_copy(0, 0, sbuf).start()
            gather_copy(1, 1, sbuf).start()
            build_rows(0, 0, sbuf)
            build_rows(1, 1, sbuf)

            def body(q, carry):
                for b in range(NBUF):
                    c = q * NBUF + b
                    gather_copy(c, b, sbuf).wait()
                    scatter_copy(gbase + c, b).start()
                    nxt = c + 2
                    bn = (b + 2) % NBUF

                    @pl.when(nxt < SB)
                    def _():
                        @pl.when(nxt >= NBUF)
                        def _():
                            scatter_copy(gbase + nxt - NBUF, bn).wait()

                        gather_copy(nxt, bn, sbuf).start()
                        build_rows(nxt, bn, sbuf)

                return carry

            lax.fori_loop(0, SB // NBUF, body, 0)
            for b in range(NBUF):
                scatter_copy(gbase + SB - NBUF + b, b).wait()

            @pl.when(sb + 1 < n_sb)
            def _():
                idx_copy(sb + 1, 1 - sbuf).wait()

        def outer(sp, carry):
            run_sb(sp * 2, 0)
            run_sb(sp * 2 + 1, 1)
            return carry

        lax.fori_loop(0, n_sb // 2, outer, 0)

    return k(idx2d, table, table_t_flat)


def kernel(tokens, table):
    b, t = tokens.shape
    b_total = b * t
    idx2d = tokens.reshape(b_total // 128, 128).astype(jnp.int32)
    table_t_flat = table.T.reshape(-1)
    out = _sc_embed(idx2d, table, table_t_flat, b_total)
    return out.reshape(b, t, DIM)


# speedup vs baseline: 4.8063x; 1.7816x over previous
import functools

import jax
import jax.numpy as jnp
from jax import lax
from jax.experimental import pallas as pl
from jax.experimental.pallas import tpu as pltpu
from jax.experimental.pallas import tpu_sc as plsc

NUM_ATOMIC = 119
DIM = 128
NC, NS = 2, 16
NW = NC * NS

CHUNK = 128
NBUF = 4
SB = 80


@functools.partial(jax.jit, static_argnames=("b_total",))
def _sc_embed(idx2d, table, b_total):
    b_per_w = b_total // NW
    n_chunks = b_per_w // CHUNK
    n_sb = n_chunks // SB
    rows_per_w = b_per_w // 128

    mesh = plsc.VectorSubcoreMesh(core_axis_name="c", subcore_axis_name="s")

    @functools.partial(
        pl.kernel,
        mesh=mesh,
        out_type=jax.ShapeDtypeStruct((b_total, DIM), jnp.float32),
        scratch_types=[
            pltpu.VMEM((2, SB, 128), jnp.int32),
            pltpu.VMEM((NBUF, CHUNK, DIM), jnp.float32),
            pltpu.VMEM_SHARED((NUM_ATOMIC, DIM), jnp.float32),
            [pltpu.SemaphoreType.DMA] * NBUF,
            pltpu.SemaphoreType.DMA,
        ],
    )
    def k(idx_hbm, table_hbm, out_hbm, idx_sb, rows_v, table_sp, ssem, isem):
        wid = lax.axis_index("s") * NC + lax.axis_index("c")
        row_base = wid * rows_per_w
        tok_base = wid * b_per_w

        @pl.when(lax.axis_index("s") == 0)
        def _():
            pltpu.sync_copy(table_hbm, table_sp)

        plsc.subcore_barrier()

        def idx_copy(sb, sbuf):
            return pltpu.make_async_copy(
                idx_hbm.at[pl.ds(row_base + sb * SB, SB)], idx_sb.at[sbuf], isem
            )

        def gather_copy(c, b, sbuf):
            return pltpu.make_async_copy(
                table_sp.at[idx_sb.at[sbuf, c]], rows_v.at[b], ssem[b]
            )

        def scatter_copy(g, b):
            return pltpu.make_async_copy(
                rows_v.at[b], out_hbm.at[pl.ds(tok_base + g * CHUNK, CHUNK)], ssem[b]
            )

        idx_copy(0, 0).start()
        idx_copy(0, 0).wait()

        def run_sb(sb, sbuf):
            gbase = sb * SB

            @pl.when(sb + 1 < n_sb)
            def _():
                idx_copy(sb + 1, 1 - sbuf).start()

            gather_copy(0, 0, sbuf).start()
            gather_copy(1, 1, sbuf).start()

            def body(q, carry):
                for b in range(NBUF):
                    c = q * NBUF + b
                    gather_copy(c, b, sbuf).wait()
                    scatter_copy(gbase + c, b).start()
                    nxt = c + 2
                    bn = (b + 2) % NBUF

                    @pl.when(nxt < SB)
                    def _():
                        @pl.when(nxt >= NBUF)
                        def _():
                            scatter_copy(gbase + nxt - NBUF, bn).wait()

                        gather_copy(nxt, bn, sbuf).start()

                return carry

            lax.fori_loop(0, SB // NBUF, body, 0)
            for b in range(NBUF):
                scatter_copy(gbase + SB - NBUF + b, b).wait()

            @pl.when(sb + 1 < n_sb)
            def _():
                idx_copy(sb + 1, 1 - sbuf).wait()

        def outer(sp, carry):
            run_sb(sp * 2, 0)
            run_sb(sp * 2 + 1, 1)
            return carry

        lax.fori_loop(0, n_sb // 2, outer, 0)

    return k(idx2d, table)


def kernel(tokens, table):
    b, t = tokens.shape
    b_total = b * t
    idx2d = tokens.reshape(b_total // 128, 128).astype(jnp.int32)
    out = _sc_embed(idx2d, table, b_total)
    return out.reshape(b, t, DIM)
